# Initial kernel scaffold; baseline (speedup 1.0000x reference)
import jax
import jax.numpy as jnp
from jax.experimental import pallas as pl


def kernel(features, edge_index, W1, b1, W2, b2, W3, b3, W4, b4, W5, b5):
    raise NotImplementedError("write your pallas kernel here")


# speedup vs baseline: 2.9583x; 2.9583x over previous
import functools

import jax
import jax.numpy as jnp
from jax import lax
from jax.experimental import pallas as pl
from jax.experimental.pallas import tpu as pltpu
from jax.experimental.pallas import tpu_sc as plsc

N = 10000
E = 320000
NIN = 128
NH = 256
NOUT = 128

NSUB = 16
KCH = 1024
EPW = 20480
EP = EPW * NSUB
NCHUNKS = EPW // KCH
ACC_ROWS = 10016
PROWS = 624


def _make_mp(split_cols):
  D = 128
  mesh = plsc.VectorSubcoreMesh(core_axis_name="c", subcore_axis_name="s")

  @functools.partial(
      pl.kernel,
      out_type=jax.ShapeDtypeStruct((2 * N, D), jnp.float32),
      mesh=mesh,
      scratch_types=[
          pltpu.VMEM((256, D), jnp.float32),
          pltpu.VMEM((8, 128), jnp.int32),
          pltpu.VMEM((8, 128), jnp.int32),
          pltpu.VMEM((8, 128), jnp.int32),
          pltpu.VMEM_SHARED((ACC_ROWS, D), jnp.float32),
          pltpu.SemaphoreType.DMA,
      ],
  )
  def mp(table, srcm, dstm, out, rows_v, src_v, gidx_v, dst_v, acc, sem):
    c = lax.axis_index("c")
    s = lax.axis_index("s")
    zero16 = jnp.zeros((16,), jnp.float32)

    def zrow(r, carry):
      for l in range(D // 16):
        rows_v[r, pl.ds(l * 16, 16)] = zero16
      return carry
    lax.fori_loop(0, 256, zrow, 0)
    zbase = s * PROWS
    for off, sz in ((0, 256), (256, 256), (512, PROWS - 512)):
      pltpu.sync_copy(rows_v.at[pl.ds(0, sz)], acc.at[pl.ds(zbase + off, sz)])

    @pl.when(s == 0)
    def _zero_tail():
      pltpu.sync_copy(rows_v.at[pl.ds(0, ACC_ROWS - NSUB * PROWS)],
                      acc.at[pl.ds(NSUB * PROWS, ACC_ROWS - NSUB * PROWS)])
    plsc.subcore_barrier()

    if split_cols:
      ebase = s * (EPW // 128)
      nchunks = NCHUNKS
    else:
      ebase = (s * 2 + c) * (EPW // 256)
      nchunks = NCHUNKS // 2

    def chunk(j, carry):
      r0 = ebase + j * (KCH // 128)
      pltpu.sync_copy(srcm.at[pl.ds(r0, 8)], src_v)
      pltpu.sync_copy(dstm.at[pl.ds(r0, 8)], dst_v)
      if split_cols:
        for r in range(8):
          for l in range(8):
            v = src_v[r, pl.ds(l * 16, 16)]
            gidx_v[r, pl.ds(l * 16, 16)] = v * 2 + c
        idx = gidx_v
      else:
        idx = src_v
      for jj in range(8):
        pltpu.async_copy(
            table.at[idx.at[jj]], rows_v.at[pl.ds(0, 128)], sem
        ).wait()
        pltpu.sync_copy(
            rows_v.at[pl.ds(0, 128)], acc.at[dst_v.at[jj]], add=True
        )
      return carry
    lax.fori_loop(0, nchunks, chunk, 0)
    plsc.subcore_barrier()

    wbase = s * PROWS
    for off, sz in ((0, 256), (256, 256), (512, PROWS - 512)):
      pltpu.sync_copy(acc.at[pl.ds(wbase + off, sz)], rows_v.at[pl.ds(0, sz)])
      pltpu.sync_copy(
          rows_v.at[pl.ds(0, sz)], out.at[pl.ds(c * N + wbase + off, sz)]
      )

    @pl.when(s == 0)
    def _wb_tail():
      tail = N - NSUB * PROWS
      pltpu.sync_copy(acc.at[pl.ds(NSUB * PROWS, tail)],
                      rows_v.at[pl.ds(0, tail)])
      pltpu.sync_copy(rows_v.at[pl.ds(0, tail)],
                      out.at[pl.ds(c * N + NSUB * PROWS, tail)])

  return mp


_mp_cols = _make_mp(True)
_mp_edges = _make_mp(False)


def _mm1_body(x_ref, w_ref, o_ref):
  o_ref[...] = jnp.dot(x_ref[...], w_ref[...],
                       preferred_element_type=jnp.float32)


def _mid_body(a_ref, b_ref, w_ref, o_ref):
  h0 = jnp.maximum(a_ref[0:N, :] + b_ref[0:1, :], 0.0)
  h1 = jnp.maximum(a_ref[N:2 * N, :] + b_ref[1:2, :], 0.0)
  o_ref[...] = (
      jnp.dot(h0, w_ref[0:128, :], preferred_element_type=jnp.float32)
      + jnp.dot(h1, w_ref[128:256, :], preferred_element_type=jnp.float32))


def _mid_fh_body(a_ref, b_ref, w_ref, o_ref, fh_ref):
  f0 = a_ref[0:N, :] + b_ref[0:1, :]
  f1 = a_ref[N:2 * N, :] + b_ref[1:2, :]
  fh_ref[:, 0:128] = f0
  fh_ref[:, 128:256] = f1
  h0 = jnp.maximum(f0, 0.0)
  h1 = jnp.maximum(f1, 0.0)
  o_ref[...] = (
      jnp.dot(h0, w_ref[0:128, :], preferred_element_type=jnp.float32)
      + jnp.dot(h1, w_ref[128:256, :], preferred_element_type=jnp.float32))


def _fin_body(a_ref, b_ref, o_ref):
  fo = a_ref[0:N, :] + a_ref[N:2 * N, :] + b_ref[0:1, :]
  m = jnp.max(fo, axis=0, keepdims=True)
  ex = jnp.exp(fo - m)
  lse = jnp.log(jnp.sum(ex, axis=0, keepdims=True))
  o_ref[...] = fo - m - lse


_mm1 = pl.pallas_call(
    _mm1_body, out_shape=jax.ShapeDtypeStruct((N, NH), jnp.float32))
_mid = pl.pallas_call(
    _mid_body, out_shape=jax.ShapeDtypeStruct((N, NH), jnp.float32))
_mid5 = pl.pallas_call(
    _mid_body, out_shape=jax.ShapeDtypeStruct((N, NOUT), jnp.float32))
_mid_fh = pl.pallas_call(
    _mid_fh_body,
    out_shape=(jax.ShapeDtypeStruct((N, NOUT), jnp.float32),
               jax.ShapeDtypeStruct((N, NH), jnp.float32)))
_fin = pl.pallas_call(
    _fin_body, out_shape=jax.ShapeDtypeStruct((N, NOUT), jnp.float32))


def kernel(features, edge_index, W1, b1, W2, b2, W3, b3, W4, b4, W5, b5):
  src = edge_index[0]
  dst = edge_index[1]
  pad = EP - E
  srcm = jnp.concatenate(
      [src, jnp.zeros((pad,), jnp.int32)]).reshape(EP // 128, 128)
  dstm = jnp.concatenate(
      [dst, jnp.full((pad,), N, jnp.int32)]).reshape(EP // 128, 128)

  t1 = _mm1(features, W1)
  a1 = _mp_cols(t1.reshape(2 * N, 128), srcm, dstm)
  t2 = _mid(a1, b1.reshape(2, 128), W2)
  a2 = _mp_cols(t2.reshape(2 * N, 128), srcm, dstm)
  t3 = _mid(a2, b2.reshape(2, 128), W3)
  a3 = _mp_cols(t3.reshape(2 * N, 128), srcm, dstm)
  t4 = _mid(a3, b3.reshape(2, 128), W4)
  a4 = _mp_cols(t4.reshape(2 * N, 128), srcm, dstm)
  t5, features_hidden = _mid_fh(a4, b4.reshape(2, 128), W5)
  a5 = _mp_edges(t5, srcm, dstm)
  probs = _fin(a5, b5.reshape(1, 128))
  return (probs, features_hidden)


# speedup vs baseline: 3.6410x; 1.2308x over previous
import functools

import jax
import jax.numpy as jnp
from jax import lax
from jax.experimental import pallas as pl
from jax.experimental.pallas import tpu as pltpu
from jax.experimental.pallas import tpu_sc as plsc

N = 10000
E = 320000
NIN = 128
NH = 256
NOUT = 128

NSUB = 16
EPW = 20480
EP = EPW * NSUB
ACC_ROWS = 10008
PROWS = 624
NSLOT = 2
CH = 16


def _make_mp(split_cols):
  D = 128
  if split_cols:
    nrows = EPW // 128
  else:
    nrows = EPW // 256
  nch = nrows // CH
  mesh = plsc.VectorSubcoreMesh(core_axis_name="c", subcore_axis_name="s")

  @functools.partial(
      pl.kernel,
      out_type=jax.ShapeDtypeStruct((2 * N, D), jnp.float32),
      mesh=mesh,
      scratch_types=[
          pltpu.VMEM((NSLOT * 128, D), jnp.float32),
          pltpu.VMEM((2, CH, 128), jnp.int32),
          pltpu.VMEM((2, CH, 128), jnp.int32),
          pltpu.VMEM_SHARED((ACC_ROWS, D), jnp.float32),
          pltpu.SemaphoreType.DMA,
          pltpu.SemaphoreType.DMA,
      ],
  )
  def mp(table, gsrcm, dstm, out, rows_v, gidx_v, dst_v, acc, semg, semi):
    c = lax.axis_index("c")
    s = lax.axis_index("s")
    zero16 = jnp.zeros((16,), jnp.float32)

    if split_cols:
      gbase = c * (EP // 128) + s * nrows
      dbase = s * nrows
    else:
      gbase = (s * 2 + c) * nrows
      dbase = gbase

    def idx_cp(jc, buf):
      return (
          pltpu.async_copy(
              gsrcm.at[pl.ds(gbase + jc * CH, CH)], gidx_v.at[buf], semi),
          pltpu.async_copy(
              dstm.at[pl.ds(dbase + jc * CH, CH)], dst_v.at[buf], semi),
      )

    def gather_cp(buf, row, slot):
      return pltpu.make_async_copy(
          table.at[gidx_v.at[buf, row]],
          rows_v.at[pl.ds(slot * 128, 128)], semg)

    pre_g, pre_d = idx_cp(jnp.int32(0), jnp.int32(0))

    def zrow(r, carry):
      for l in range(D // 16):
        rows_v[r, pl.ds(l * 16, 16)] = zero16
      return carry
    lax.fori_loop(0, NSLOT * 128, zrow, 0)
    zbase = s * PROWS
    for off, sz in ((0, 256), (256, 256), (512, PROWS - 512)):
      pltpu.sync_copy(rows_v.at[pl.ds(0, sz)], acc.at[pl.ds(zbase + off, sz)])

    @pl.when(s == 0)
    def _zero_tail():
      pltpu.sync_copy(rows_v.at[pl.ds(0, ACC_ROWS - NSUB * PROWS)],
                      acc.at[pl.ds(NSUB * PROWS, ACC_ROWS - NSUB * PROWS)])
    pre_g.wait()
    pre_d.wait()
    plsc.subcore_barrier()

    gather_cp(jnp.int32(0), jnp.int32(0), 0).start()
    gather_cp(jnp.int32(0), jnp.int32(1), 1).start()

    def chunk_body(jc, carry):
      cur = lax.rem(jc, 2)
      nxt = lax.rem(jc + 1, 2)

      @pl.when(jc + 1 < nch)
      def _prefetch_idx():
        idx_cp(jc + 1, nxt)

      for jj in range(CH):
        slot = jj % NSLOT
        gather_cp(cur, jnp.int32(jj), slot).wait()
        pltpu.sync_copy(
            rows_v.at[pl.ds(slot * 128, 128)],
            acc.at[dst_v.at[cur, jj]], add=True)
        if jj < CH - NSLOT:
          gather_cp(cur, jnp.int32(jj + NSLOT), slot).start()
        else:
          nrow = jj - (CH - NSLOT)

          @pl.when(jc + 1 < nch)
          def _fire_next_chunk():
            if nrow == 0:
              pltpu.make_async_copy(
                  gsrcm.at[pl.ds(gbase + (jc + 1) * CH, CH)],
                  gidx_v.at[nxt], semi).wait()
              pltpu.make_async_copy(
                  dstm.at[pl.ds(dbase + (jc + 1) * CH, CH)],
                  dst_v.at[nxt], semi).wait()
            gather_cp(nxt, jnp.int32(nrow), slot).start()
      return carry
    lax.fori_loop(0, nch, chunk_body, 0)
    plsc.subcore_barrier()

    wbase = s * PROWS
    for off, sz in ((0, 256), (256, 256), (512, PROWS - 512)):
      pltpu.sync_copy(acc.at[pl.ds(wbase + off, sz)], rows_v.at[pl.ds(0, sz)])
      pltpu.sync_copy(
          rows_v.at[pl.ds(0, sz)], out.at[pl.ds(c * N + wbase + off, sz)]
      )

    @pl.when(s == 0)
    def _wb_tail():
      tail = N - NSUB * PROWS
      pltpu.sync_copy(acc.at[pl.ds(NSUB * PROWS, tail)],
                      rows_v.at[pl.ds(0, tail)])
      pltpu.sync_copy(rows_v.at[pl.ds(0, tail)],
                      out.at[pl.ds(c * N + NSUB * PROWS, tail)])

  return mp


_mp_cols = _make_mp(True)
_mp_edges = _make_mp(False)


def _mm1_body(x_ref, w_ref, o_ref):
  o_ref[...] = jnp.dot(x_ref[...], w_ref[...],
                       preferred_element_type=jnp.float32)


def _mid_body(a_ref, b_ref, w_ref, o_ref):
  h0 = jnp.maximum(a_ref[0:N, :] + b_ref[0:1, :], 0.0)
  h1 = jnp.maximum(a_ref[N:2 * N, :] + b_ref[1:2, :], 0.0)
  o_ref[...] = (
      jnp.dot(h0, w_ref[0:128, :], preferred_element_type=jnp.float32)
      + jnp.dot(h1, w_ref[128:256, :], preferred_element_type=jnp.float32))


def _mid_fh_body(a_ref, b_ref, w_ref, o_ref, fh_ref):
  f0 = a_ref[0:N, :] + b_ref[0:1, :]
  f1 = a_ref[N:2 * N, :] + b_ref[1:2, :]
  fh_ref[:, 0:128] = f0
  fh_ref[:, 128:256] = f1
  h0 = jnp.maximum(f0, 0.0)
  h1 = jnp.maximum(f1, 0.0)
  o_ref[...] = (
      jnp.dot(h0, w_ref[0:128, :], preferred_element_type=jnp.float32)
      + jnp.dot(h1, w_ref[128:256, :], preferred_element_type=jnp.float32))


def _fin_body(a_ref, b_ref, o_ref):
  fo = a_ref[0:N, :] + a_ref[N:2 * N, :] + b_ref[0:1, :]
  m = jnp.max(fo, axis=0, keepdims=True)
  ex = jnp.exp(fo - m)
  lse = jnp.log(jnp.sum(ex, axis=0, keepdims=True))
  o_ref[...] = fo - m - lse


_mm1 = pl.pallas_call(
    _mm1_body, out_shape=jax.ShapeDtypeStruct((N, NH), jnp.float32))
_mid = pl.pallas_call(
    _mid_body, out_shape=jax.ShapeDtypeStruct((N, NH), jnp.float32))
_mid5 = pl.pallas_call(
    _mid_body, out_shape=jax.ShapeDtypeStruct((N, NOUT), jnp.float32))
_mid_fh = pl.pallas_call(
    _mid_fh_body,
    out_shape=(jax.ShapeDtypeStruct((N, NOUT), jnp.float32),
               jax.ShapeDtypeStruct((N, NH), jnp.float32)))
_fin = pl.pallas_call(
    _fin_body, out_shape=jax.ShapeDtypeStruct((N, NOUT), jnp.float32))


def kernel(features, edge_index, W1, b1, W2, b2, W3, b3, W4, b4, W5, b5):
  src = edge_index[0]
  dst = edge_index[1]
  pad = EP - E
  srcp = jnp.concatenate([src, jnp.zeros((pad,), jnp.int32)])
  g0 = srcp * 2
  gsrcm = jnp.concatenate([g0, g0 + 1]).reshape(2 * EP // 128, 128)
  srcm = srcp.reshape(EP // 128, 128)
  dstm = jnp.concatenate(
      [dst, jnp.full((pad,), N, jnp.int32)]).reshape(EP // 128, 128)

  t1 = _mm1(features, W1)
  a1 = _mp_cols(t1.reshape(2 * N, 128), gsrcm, dstm)
  t2 = _mid(a1, b1.reshape(2, 128), W2)
  a2 = _mp_cols(t2.reshape(2 * N, 128), gsrcm, dstm)
  t3 = _mid(a2, b2.reshape(2, 128), W3)
  a3 = _mp_cols(t3.reshape(2 * N, 128), gsrcm, dstm)
  t4 = _mid(a3, b3.reshape(2, 128), W4)
  a4 = _mp_cols(t4.reshape(2 * N, 128), gsrcm, dstm)
  t5, features_hidden = _mid_fh(a4, b4.reshape(2, 128), W5)
  a5 = _mp_edges(t5, srcm, dstm)
  probs = _fin(a5, b5.reshape(1, 128))
  return (probs, features_hidden)


# speedup vs baseline: 3.6594x; 1.0051x over previous
import functools

import jax
import jax.numpy as jnp
from jax import lax
from jax.experimental import pallas as pl
from jax.experimental.pallas import tpu as pltpu
from jax.experimental.pallas import tpu_sc as plsc

N = 10000
E = 320000
NIN = 128
NH = 256
NOUT = 128

NSUB = 16
EPW = 20480
EP = EPW * NSUB
ACC_ROWS = 10008
PROWS = 624
NSLOT = 2
CH = 16


def _make_mp(split_cols):
  D = 128
  if split_cols:
    nrows = EPW // 128
  else:
    nrows = EPW // 256
  nch = nrows // CH
  mesh = plsc.VectorSubcoreMesh(core_axis_name="c", subcore_axis_name="s")

  @functools.partial(
      pl.kernel,
      out_type=jax.ShapeDtypeStruct((2 * N, D), jnp.float32),
      mesh=mesh,
      scratch_types=[
          pltpu.VMEM((NSLOT * 128, D), jnp.float32),
          pltpu.VMEM((2, CH, 128), jnp.int32),
          pltpu.VMEM((2, CH, 128), jnp.int32),
          pltpu.VMEM_SHARED((ACC_ROWS, D), jnp.float32),
          pltpu.SemaphoreType.DMA,
          pltpu.SemaphoreType.DMA,
      ],
  )
  def mp(table, gsrcm, dstm, out, rows_v, gidx_v, dst_v, acc, semg, semi):
    c = lax.axis_index("c")
    s = lax.axis_index("s")
    zero16 = jnp.zeros((16,), jnp.float32)

    if split_cols:
      gbase = c * (EP // 128) + s * nrows
      dbase = s * nrows
    else:
      gbase = (s * 2 + c) * nrows
      dbase = gbase

    def idx_cp(jc, buf):
      return (
          pltpu.async_copy(
              gsrcm.at[pl.ds(gbase + jc * CH, CH)], gidx_v.at[buf], semi),
          pltpu.async_copy(
              dstm.at[pl.ds(dbase + jc * CH, CH)], dst_v.at[buf], semi),
      )

    def gather_half(buf, row, half, slot):
      return pltpu.make_async_copy(
          table.at[gidx_v.at[buf, row, pl.ds(half * 64, 64)]],
          rows_v.at[pl.ds(slot * 128 + half * 64, 64)], semg)

    def gather_start(buf, row, slot):
      gather_half(buf, row, 0, slot).start()
      gather_half(buf, row, 1, slot).start()

    def gather_wait(buf, row, slot):
      gather_half(buf, row, 0, slot).wait()
      gather_half(buf, row, 1, slot).wait()

    pre_g, pre_d = idx_cp(jnp.int32(0), jnp.int32(0))

    def zrow(r, carry):
      for l in range(D // 16):
        rows_v[r, pl.ds(l * 16, 16)] = zero16
      return carry
    lax.fori_loop(0, NSLOT * 128, zrow, 0)
    zbase = s * PROWS
    for off, sz in ((0, 256), (256, 256), (512, PROWS - 512)):
      pltpu.sync_copy(rows_v.at[pl.ds(0, sz)], acc.at[pl.ds(zbase + off, sz)])

    @pl.when(s == 0)
    def _zero_tail():
      pltpu.sync_copy(rows_v.at[pl.ds(0, ACC_ROWS - NSUB * PROWS)],
                      acc.at[pl.ds(NSUB * PROWS, ACC_ROWS - NSUB * PROWS)])
    pre_g.wait()
    pre_d.wait()
    plsc.subcore_barrier()

    gather_start(jnp.int32(0), jnp.int32(0), 0)
    gather_start(jnp.int32(0), jnp.int32(1), 1)

    def chunk_body(jc, carry):
      cur = lax.rem(jc, 2)
      nxt = lax.rem(jc + 1, 2)

      @pl.when(jc + 1 < nch)
      def _prefetch_idx():
        idx_cp(jc + 1, nxt)

      for jj in range(CH):
        slot = jj % NSLOT
        gather_wait(cur, jnp.int32(jj), slot)
        pltpu.sync_copy(
            rows_v.at[pl.ds(slot * 128, 128)],
            acc.at[dst_v.at[cur, jj]], add=True)
        if jj < CH - NSLOT:
          gather_start(cur, jnp.int32(jj + NSLOT), slot)
        else:
          nrow = jj - (CH - NSLOT)

          @pl.when(jc + 1 < nch)
          def _fire_next_chunk():
            if nrow == 0:
              pltpu.make_async_copy(
                  gsrcm.at[pl.ds(gbase + (jc + 1) * CH, CH)],
                  gidx_v.at[nxt], semi).wait()
              pltpu.make_async_copy(
                  dstm.at[pl.ds(dbase + (jc + 1) * CH, CH)],
                  dst_v.at[nxt], semi).wait()
            gather_start(nxt, jnp.int32(nrow), slot)
      return carry
    lax.fori_loop(0, nch, chunk_body, 0)
    plsc.subcore_barrier()

    wbase = s * PROWS
    for off, sz in ((0, 256), (256, 256), (512, PROWS - 512)):
      pltpu.sync_copy(acc.at[pl.ds(wbase + off, sz)], rows_v.at[pl.ds(0, sz)])
      pltpu.sync_copy(
          rows_v.at[pl.ds(0, sz)], out.at[pl.ds(c * N + wbase + off, sz)]
      )

    @pl.when(s == 0)
    def _wb_tail():
      tail = N - NSUB * PROWS
      pltpu.sync_copy(acc.at[pl.ds(NSUB * PROWS, tail)],
                      rows_v.at[pl.ds(0, tail)])
      pltpu.sync_copy(rows_v.at[pl.ds(0, tail)],
                      out.at[pl.ds(c * N + NSUB * PROWS, tail)])

  return mp


_mp_cols = _make_mp(True)
_mp_edges = _make_mp(False)


def _mm1_body(x_ref, w_ref, o_ref):
  o_ref[...] = jnp.dot(x_ref[...], w_ref[...],
                       preferred_element_type=jnp.float32)


def _mid_body(a_ref, b_ref, w_ref, o_ref):
  h0 = jnp.maximum(a_ref[0:N, :] + b_ref[0:1, :], 0.0)
  h1 = jnp.maximum(a_ref[N:2 * N, :] + b_ref[1:2, :], 0.0)
  o_ref[...] = (
      jnp.dot(h0, w_ref[0:128, :], preferred_element_type=jnp.float32)
      + jnp.dot(h1, w_ref[128:256, :], preferred_element_type=jnp.float32))


def _mid_fh_body(a_ref, b_ref, w_ref, o_ref, fh_ref):
  f0 = a_ref[0:N, :] + b_ref[0:1, :]
  f1 = a_ref[N:2 * N, :] + b_ref[1:2, :]
  fh_ref[:, 0:128] = f0
  fh_ref[:, 128:256] = f1
  h0 = jnp.maximum(f0, 0.0)
  h1 = jnp.maximum(f1, 0.0)
  o_ref[...] = (
      jnp.dot(h0, w_ref[0:128, :], preferred_element_type=jnp.float32)
      + jnp.dot(h1, w_ref[128:256, :], preferred_element_type=jnp.float32))


def _fin_body(a_ref, b_ref, o_ref):
  fo = a_ref[0:N, :] + a_ref[N:2 * N, :] + b_ref[0:1, :]
  m = jnp.max(fo, axis=0, keepdims=True)
  ex = jnp.exp(fo - m)
  lse = jnp.log(jnp.sum(ex, axis=0, keepdims=True))
  o_ref[...] = fo - m - lse


_mm1 = pl.pallas_call(
    _mm1_body, out_shape=jax.ShapeDtypeStruct((N, NH), jnp.float32))
_mid = pl.pallas_call(
    _mid_body, out_shape=jax.ShapeDtypeStruct((N, NH), jnp.float32))
_mid5 = pl.pallas_call(
    _mid_body, out_shape=jax.ShapeDtypeStruct((N, NOUT), jnp.float32))
_mid_fh = pl.pallas_call(
    _mid_fh_body,
    out_shape=(jax.ShapeDtypeStruct((N, NOUT), jnp.float32),
               jax.ShapeDtypeStruct((N, NH), jnp.float32)))
_fin = pl.pallas_call(
    _fin_body, out_shape=jax.ShapeDtypeStruct((N, NOUT), jnp.float32))


def kernel(features, edge_index, W1, b1, W2, b2, W3, b3, W4, b4, W5, b5):
  src = edge_index[0]
  dst = edge_index[1]
  pad = EP - E
  srcp = jnp.concatenate([src, jnp.zeros((pad,), jnp.int32)])
  g0 = srcp * 2
  gsrcm = jnp.concatenate([g0, g0 + 1]).reshape(2 * EP // 128, 128)
  srcm = srcp.reshape(EP // 128, 128)
  dstm = jnp.concatenate(
      [dst, jnp.full((pad,), N, jnp.int32)]).reshape(EP // 128, 128)

  t1 = _mm1(features, W1)
  a1 = _mp_cols(t1.reshape(2 * N, 128), gsrcm, dstm)
  t2 = _mid(a1, b1.reshape(2, 128), W2)
  a2 = _mp_cols(t2.reshape(2 * N, 128), gsrcm, dstm)
  t3 = _mid(a2, b2.reshape(2, 128), W3)
  a3 = _mp_cols(t3.reshape(2 * N, 128), gsrcm, dstm)
  t4 = _mid(a3, b3.reshape(2, 128), W4)
  a4 = _mp_cols(t4.reshape(2 * N, 128), gsrcm, dstm)
  t5, features_hidden = _mid_fh(a4, b4.reshape(2, 128), W5)
  a5 = _mp_edges(t5, srcm, dstm)
  probs = _fin(a5, b5.reshape(1, 128))
  return (probs, features_hidden)
